# Initial kernel scaffold; baseline (speedup 1.0000x reference)
import jax
import jax.numpy as jnp
from jax import lax
from jax.experimental import pallas as pl
from jax.experimental.pallas import tpu as pltpu
from jax.experimental.pallas import tpu_sc as plsc

_N = 10000
_E = 320000
_F = 2048
_M = 256
_D = 128
_H = 256
_NE = 8
_NT = 3
_NC = 2
_NS = 16
_EC = 125
_ECH = 80
_NPT = _N // _NS
_NP = 10240
_PC = 64
_PCH = 5
_FPT = _F // _NS
_CW = 16
_BN_SCALE = 1.0 / (1.0 + 1e-5) ** 0.5
_PREC = lax.Precision.HIGHEST


def _edge_agg_body(h_hbm, idx_hbm, z_hbm, out_hbm, src_v, dst_v, rows_v, acc_sh, sem):
    c = lax.axis_index("c")
    s = lax.axis_index("s")
    pltpu.sync_copy(z_hbm, acc_sh.at[pl.ds(s * _NPT, _NPT)])
    pltpu.sync_copy(idx_hbm.at[0, c, s], src_v)
    pltpu.sync_copy(idx_hbm.at[1, c, s], dst_v)
    plsc.subcore_barrier()

    def _step(j, carry):
        pltpu.async_copy(h_hbm.at[src_v.at[j]], rows_v, sem).wait()
        pltpu.sync_copy(rows_v, acc_sh.at[dst_v.at[j]], add=True)
        return carry

    lax.fori_loop(0, _ECH, _step, 0)
    plsc.subcore_barrier()
    pltpu.sync_copy(acc_sh.at[pl.ds(s * _NPT, _NPT)],
                    out_hbm.at[c, pl.ds(s * _NPT, _NPT)])


_edge_agg = pl.kernel(
    _edge_agg_body,
    out_type=jax.ShapeDtypeStruct((_NC, _N, _D), jnp.float32),
    mesh=plsc.VectorSubcoreMesh(core_axis_name="c", subcore_axis_name="s"),
    scratch_types=[
        pltpu.VMEM((_ECH, _EC), jnp.int32),
        pltpu.VMEM((_ECH, _EC), jnp.int32),
        pltpu.VMEM((_EC, _D), jnp.float32),
        pltpu.VMEM_SHARED((_N, _D), jnp.float32),
        pltpu.SemaphoreType.DMA,
    ],
)


def _pool_body(h_hbm, bidx_hbm, cval_hbm, zs_hbm, zc_hbm, sum_hbm, cnt_hbm,
               idx_v, rows_v, cval_v, acc_sh, cacc_sh):
    c = lax.axis_index("c")
    s = lax.axis_index("s")
    w = c * _NS + s
    pltpu.sync_copy(zs_hbm, acc_sh.at[pl.ds(s * _FPT, _FPT)])
    pltpu.sync_copy(zc_hbm, cacc_sh.at[pl.ds(s * _FPT, _FPT)])
    pltpu.sync_copy(bidx_hbm.at[c, s], idx_v)
    plsc.subcore_barrier()
    base = w * (_PC * _PCH)

    def _step(j, carry):
        pltpu.sync_copy(h_hbm.at[pl.ds(base + j * _PC, _PC)], rows_v)
        pltpu.sync_copy(cval_hbm.at[pl.ds(base + j * _PC, _PC)], cval_v)
        pltpu.sync_copy(rows_v, acc_sh.at[idx_v.at[j]], add=True)
        pltpu.sync_copy(cval_v, cacc_sh.at[idx_v.at[j]], add=True)
        return carry

    lax.fori_loop(0, _PCH, _step, 0)
    plsc.subcore_barrier()
    pltpu.sync_copy(acc_sh.at[pl.ds(s * _FPT, _FPT)],
                    sum_hbm.at[c, pl.ds(s * _FPT, _FPT)])
    pltpu.sync_copy(cacc_sh.at[pl.ds(s * _FPT, _FPT)],
                    cnt_hbm.at[c, pl.ds(s * _FPT, _FPT)])


_pool = pl.kernel(
    _pool_body,
    out_type=(jax.ShapeDtypeStruct((_NC, _F, _D), jnp.float32),
              jax.ShapeDtypeStruct((_NC, _F, _CW), jnp.float32)),
    mesh=plsc.VectorSubcoreMesh(core_axis_name="c", subcore_axis_name="s"),
    scratch_types=[
        pltpu.VMEM((_PCH, _PC), jnp.int32),
        pltpu.VMEM((_PC, _D), jnp.float32),
        pltpu.VMEM((_PC, _CW), jnp.float32),
        pltpu.VMEM_SHARED((_F, _D), jnp.float32),
        pltpu.VMEM_SHARED((_F, _CW), jnp.float32),
    ],
)


def _gin_body(eps_ref, x_ref, a_ref, b_ref, w1_ref, b1_ref, w2_ref, b2_ref, o_ref):
    m = (1.0 + eps_ref[0, 0]) * x_ref[...] + a_ref[...] + b_ref[...]
    z = jnp.dot(m, w1_ref[...], precision=_PREC, preferred_element_type=jnp.float32)
    z = jnp.maximum(z + b1_ref[...], 0.0)
    z = jnp.dot(z, w2_ref[...], precision=_PREC, preferred_element_type=jnp.float32)
    z = (z + b2_ref[...]) * _BN_SCALE
    o_ref[...] = jnp.maximum(z, 0.0)


_GIN_BLK = 1000
_gin_call = pl.pallas_call(
    _gin_body,
    grid=(_N // _GIN_BLK,),
    in_specs=[
        pl.BlockSpec((1, 1), lambda i: (0, 0)),
        pl.BlockSpec((_GIN_BLK, _D), lambda i: (i, 0)),
        pl.BlockSpec((_GIN_BLK, _D), lambda i: (i, 0)),
        pl.BlockSpec((_GIN_BLK, _D), lambda i: (i, 0)),
        pl.BlockSpec((_D, _D), lambda i: (0, 0)),
        pl.BlockSpec((1, _D), lambda i: (0, 0)),
        pl.BlockSpec((_D, _D), lambda i: (0, 0)),
        pl.BlockSpec((1, _D), lambda i: (0, 0)),
    ],
    out_specs=pl.BlockSpec((_GIN_BLK, _D), lambda i: (i, 0)),
    out_shape=jax.ShapeDtypeStruct((_N, _D), jnp.float32),
)


def _gelu(z):
    return 0.5 * z * (1.0 + lax.erf(z * 0.7071067811865476))


def _moe_body(sum_ref, cnt_ref, gate_ref, w1_ref, b1_ref, w2_ref, b2_ref,
              w3_ref, b3_ref, mol_ref, hw1_ref, hb1_ref, hw2_ref, hb2_ref,
              preds_ref, lb_ref):
    fs = sum_ref[0] + sum_ref[1]
    cnt = cnt_ref[0, :, 0:1] + cnt_ref[1, :, 0:1]
    femb = fs / jnp.maximum(cnt, 1.0)

    logits = jnp.dot(femb, gate_ref[...], precision=_PREC,
                     preferred_element_type=jnp.float32)
    iota = lax.broadcasted_iota(jnp.int32, (_F, _NE), 1)
    v1 = jnp.max(logits, axis=1, keepdims=True)
    i1 = jnp.min(jnp.where(logits == v1, iota, _NE), axis=1, keepdims=True)
    l2 = jnp.where(iota == i1, -jnp.inf, logits)
    v2 = jnp.max(l2, axis=1, keepdims=True)
    i2 = jnp.min(jnp.where(l2 == v2, iota, _NE), axis=1, keepdims=True)
    e2 = jnp.exp(v2 - v1)
    denom = 1.0 + e2
    wts = (jnp.where(iota == i1, 1.0, 0.0)
           + jnp.where(iota == i2, e2, 0.0)) / denom
    load = jnp.sum(wts, axis=0, keepdims=True) / float(_F)
    lb_ref[0, 0] = float(_NE) * jnp.sum(load * load)

    moe = jnp.zeros((_F, _D), jnp.float32)
    for e in range(_NE):
        z = jnp.dot(femb, w1_ref[e], precision=_PREC,
                    preferred_element_type=jnp.float32) + b1_ref[e][None, :]
        z = _gelu(z)
        z = jnp.dot(z, w2_ref[e], precision=_PREC,
                    preferred_element_type=jnp.float32) + b2_ref[e][None, :]
        z = _gelu(z)
        z = jnp.dot(z, w3_ref[e], precision=_PREC,
                    preferred_element_type=jnp.float32) + b3_ref[e][None, :]
        moe = moe + wts[:, e:e + 1] * z

    iota_m = lax.broadcasted_iota(jnp.int32, (_F, _M), 1)
    sel = jnp.where(mol_ref[...] == iota_m, 1.0, 0.0)
    msum = lax.dot_general(sel, moe, (((0,), (0,)), ((), ())),
                           precision=_PREC, preferred_element_type=jnp.float32)
    ones_col = jnp.ones((_F, 1), jnp.float32)
    mcnt = lax.dot_general(sel, ones_col, (((0,), (0,)), ((), ())),
                           precision=_PREC, preferred_element_type=jnp.float32)
    memb = msum / jnp.maximum(mcnt, 1.0)

    for t in range(_NT):
        z = jnp.dot(memb, hw1_ref[t], precision=_PREC,
                    preferred_element_type=jnp.float32) + hb1_ref[t][None, :]
        z = jnp.maximum(z, 0.0)
        p = jnp.dot(z, hw2_ref[t], precision=_PREC,
                    preferred_element_type=jnp.float32) + hb2_ref[t][None, :]
        preds_ref[:, t:t + 1] = p


_moe_call = pl.pallas_call(
    _moe_body,
    out_shape=(jax.ShapeDtypeStruct((_M, _NT), jnp.float32),
               jax.ShapeDtypeStruct((1, 1), jnp.float32)),
)


def kernel(x, edge_index, batch, mol_idx, gin_W1, gin_b1, gin_W2, gin_b2,
           gin_eps, gate_W, exp_W1, exp_b1, exp_W2, exp_b2, exp_W3, exp_b3,
           head_W1, head_b1, head_W2, head_b2):
    idx5 = edge_index.astype(jnp.int32).reshape(2, _NC, _NS, _ECH, _EC)
    z625 = jnp.zeros((_NPT, _D), jnp.float32)
    h = x
    for i in range(3):
        agg = _edge_agg(h, idx5, z625)
        h = _gin_call(gin_eps[i].reshape(1, 1), h, agg[0], agg[1],
                      gin_W1[i], gin_b1[i].reshape(1, _D),
                      gin_W2[i], gin_b2[i].reshape(1, _D))
    hp = jnp.pad(h, ((0, _NP - _N), (0, 0)))
    bidx = jnp.pad(batch.astype(jnp.int32), (0, _NP - _N)).reshape(_NC, _NS, _PCH, _PC)
    cval = jnp.pad(jnp.ones((_N, _CW), jnp.float32), ((0, _NP - _N), (0, 0)))
    zs = jnp.zeros((_FPT, _D), jnp.float32)
    zc = jnp.zeros((_FPT, _CW), jnp.float32)
    fsum, fcnt = _pool(hp, bidx, cval, zs, zc)
    preds, lb = _moe_call(fsum, fcnt, gate_W, exp_W1, exp_b1, exp_W2, exp_b2,
                          exp_W3, exp_b3, mol_idx.astype(jnp.int32).reshape(_F, 1),
                          head_W1, head_b1, head_W2, head_b2)
    return preds, lb[0, 0]


# speedup vs baseline: 5.7119x; 5.7119x over previous
import functools

import jax
import jax.numpy as jnp
from jax import lax
from jax.experimental import pallas as pl
from jax.experimental.pallas import tpu as pltpu
from jax.experimental.pallas import tpu_sc as plsc

_N = 10000
_E = 320000
_F = 2048
_M = 256
_D = 128
_H = 256
_NE = 8
_NT = 3
_NC = 2
_NS = 16
_EC = 125
_ECH = 80
_NP = 10240
_NPT = _NP // _NS
_PC = 64
_PCH = 5
_FPT = _F // _NS
_CW = 128
_BN_SCALE = 1.0 / (1.0 + 1e-5) ** 0.5
_PREC = lax.Precision.HIGHEST


def _edge_agg_body(h_hbm, idx_hbm, z_hbm, out_hbm, src_v, dst_v, rows_v, acc_sh, sem):
    c = lax.axis_index("c")
    s = lax.axis_index("s")
    pltpu.sync_copy(z_hbm, acc_sh.at[pl.ds(s * _NPT, _NPT)])
    pltpu.sync_copy(idx_hbm.at[0, c, s], src_v)
    pltpu.sync_copy(idx_hbm.at[1, c, s], dst_v)
    plsc.subcore_barrier()

    def _step(j, carry):
        pltpu.async_copy(h_hbm.at[src_v.at[j]], rows_v, sem).wait()
        pltpu.sync_copy(rows_v, acc_sh.at[dst_v.at[j]], add=True)
        return carry

    lax.fori_loop(0, _ECH, _step, 0)
    plsc.subcore_barrier()
    pltpu.sync_copy(acc_sh.at[pl.ds(s * _NPT, _NPT)],
                    out_hbm.at[c, pl.ds(s * _NPT, _NPT)])


@functools.lru_cache(maxsize=None)
def _make_edge_agg():
    return pl.kernel(
        _edge_agg_body,
        out_type=jax.ShapeDtypeStruct((_NC, _NP, _D), jnp.float32),
        mesh=plsc.VectorSubcoreMesh(core_axis_name="c", subcore_axis_name="s"),
        scratch_types=[
            pltpu.VMEM((_ECH, _EC), jnp.int32),
            pltpu.VMEM((_ECH, _EC), jnp.int32),
            pltpu.VMEM((_EC, _D), jnp.float32),
            pltpu.VMEM_SHARED((_NP, _D), jnp.float32),
            pltpu.SemaphoreType.DMA,
        ],
    )


def _pool_body(h_hbm, bidx_hbm, cval_hbm, zs_hbm, zc_hbm, sum_hbm, cnt_hbm,
               idx_v, rows_v, cval_v, acc_sh, cacc_sh):
    c = lax.axis_index("c")
    s = lax.axis_index("s")
    w = c * _NS + s
    pltpu.sync_copy(zs_hbm, acc_sh.at[pl.ds(s * _FPT, _FPT)])
    pltpu.sync_copy(zc_hbm, cacc_sh.at[pl.ds(s * _FPT, _FPT)])
    pltpu.sync_copy(bidx_hbm.at[c, s], idx_v)
    plsc.subcore_barrier()
    base = w * (_PC * _PCH)

    def _step(j, carry):
        pltpu.sync_copy(h_hbm.at[pl.ds(base + j * _PC, _PC)], rows_v)
        pltpu.sync_copy(cval_hbm.at[pl.ds(base + j * _PC, _PC)], cval_v)
        pltpu.sync_copy(rows_v, acc_sh.at[idx_v.at[j]], add=True)
        pltpu.sync_copy(cval_v, cacc_sh.at[idx_v.at[j]], add=True)
        return carry

    lax.fori_loop(0, _PCH, _step, 0)
    plsc.subcore_barrier()
    pltpu.sync_copy(acc_sh.at[pl.ds(s * _FPT, _FPT)],
                    sum_hbm.at[c, pl.ds(s * _FPT, _FPT)])
    pltpu.sync_copy(cacc_sh.at[pl.ds(s * _FPT, _FPT)],
                    cnt_hbm.at[c, pl.ds(s * _FPT, _FPT)])


@functools.lru_cache(maxsize=None)
def _make_pool():
    return pl.kernel(
        _pool_body,
        out_type=(jax.ShapeDtypeStruct((_NC, _F, _D), jnp.float32),
                  jax.ShapeDtypeStruct((_NC, _F, _CW), jnp.float32)),
        mesh=plsc.VectorSubcoreMesh(core_axis_name="c", subcore_axis_name="s"),
        scratch_types=[
            pltpu.VMEM((_PCH, _PC), jnp.int32),
            pltpu.VMEM((_PC, _D), jnp.float32),
            pltpu.VMEM((_PC, _CW), jnp.float32),
            pltpu.VMEM_SHARED((_F, _D), jnp.float32),
            pltpu.VMEM_SHARED((_F, _CW), jnp.float32),
        ],
    )


def _gin_body(eps_ref, x_ref, a_ref, b_ref, w1_ref, b1_ref, w2_ref, b2_ref, o_ref):
    m = (1.0 + eps_ref[0, 0]) * x_ref[...] + a_ref[...] + b_ref[...]
    z = jnp.dot(m, w1_ref[...], precision=_PREC, preferred_element_type=jnp.float32)
    z = jnp.maximum(z + b1_ref[...], 0.0)
    z = jnp.dot(z, w2_ref[...], precision=_PREC, preferred_element_type=jnp.float32)
    z = (z + b2_ref[...]) * _BN_SCALE
    o_ref[...] = jnp.maximum(z, 0.0)


_GIN_BLK = 1000
_gin_call = pl.pallas_call(
    _gin_body,
    grid=(_N // _GIN_BLK,),
    in_specs=[
        pl.BlockSpec((1, 1), lambda i: (0, 0)),
        pl.BlockSpec((_GIN_BLK, _D), lambda i: (i, 0)),
        pl.BlockSpec((_GIN_BLK, _D), lambda i: (i, 0)),
        pl.BlockSpec((_GIN_BLK, _D), lambda i: (i, 0)),
        pl.BlockSpec((_D, _D), lambda i: (0, 0)),
        pl.BlockSpec((1, _D), lambda i: (0, 0)),
        pl.BlockSpec((_D, _D), lambda i: (0, 0)),
        pl.BlockSpec((1, _D), lambda i: (0, 0)),
    ],
    out_specs=pl.BlockSpec((_GIN_BLK, _D), lambda i: (i, 0)),
    out_shape=jax.ShapeDtypeStruct((_N, _D), jnp.float32),
)


def _gelu(z):
    return 0.5 * z * (1.0 + lax.erf(z * 0.7071067811865476))


def _moe_body(sum_ref, cnt_ref, gate_ref, w1_ref, b1_ref, w2_ref, b2_ref,
              w3_ref, b3_ref, mol_ref, hw1_ref, hb1_ref, hw2_ref, hb2_ref,
              preds_ref, lb_ref):
    fs = sum_ref[0] + sum_ref[1]
    cnt = cnt_ref[0, :, 0:1] + cnt_ref[1, :, 0:1]
    femb = fs / jnp.maximum(cnt, 1.0)

    logits = jnp.dot(femb, gate_ref[...], precision=_PREC,
                     preferred_element_type=jnp.float32)
    iota = lax.broadcasted_iota(jnp.int32, (_F, _NE), 1)
    v1 = jnp.max(logits, axis=1, keepdims=True)
    i1 = jnp.min(jnp.where(logits == v1, iota, _NE), axis=1, keepdims=True)
    l2 = jnp.where(iota == i1, -jnp.inf, logits)
    v2 = jnp.max(l2, axis=1, keepdims=True)
    i2 = jnp.min(jnp.where(l2 == v2, iota, _NE), axis=1, keepdims=True)
    e2 = jnp.exp(v2 - v1)
    denom = 1.0 + e2
    wts = (jnp.where(iota == i1, 1.0, 0.0)
           + jnp.where(iota == i2, e2, 0.0)) / denom
    load = jnp.sum(wts, axis=0, keepdims=True) / float(_F)
    lb_ref[...] = jnp.reshape(float(_NE) * jnp.sum(load * load), (1, 1))

    moe = jnp.zeros((_F, _D), jnp.float32)
    for e in range(_NE):
        z = jnp.dot(femb, w1_ref[e], precision=_PREC,
                    preferred_element_type=jnp.float32) + b1_ref[e][None, :]
        z = _gelu(z)
        z = jnp.dot(z, w2_ref[e], precision=_PREC,
                    preferred_element_type=jnp.float32) + b2_ref[e][None, :]
        z = _gelu(z)
        z = jnp.dot(z, w3_ref[e], precision=_PREC,
                    preferred_element_type=jnp.float32) + b3_ref[e][None, :]
        moe = moe + wts[:, e:e + 1] * z

    iota_m = lax.broadcasted_iota(jnp.int32, (_F, _M), 1)
    sel = jnp.where(mol_ref[...] == iota_m, 1.0, 0.0)
    msum = lax.dot_general(sel, moe, (((0,), (0,)), ((), ())),
                           precision=_PREC, preferred_element_type=jnp.float32)
    ones_col = jnp.ones((_F, 1), jnp.float32)
    mcnt = lax.dot_general(sel, ones_col, (((0,), (0,)), ((), ())),
                           precision=_PREC, preferred_element_type=jnp.float32)
    memb = msum / jnp.maximum(mcnt, 1.0)

    for t in range(_NT):
        z = jnp.dot(memb, hw1_ref[t], precision=_PREC,
                    preferred_element_type=jnp.float32) + hb1_ref[t][None, :]
        z = jnp.maximum(z, 0.0)
        p = jnp.dot(z, hw2_ref[t], precision=_PREC,
                    preferred_element_type=jnp.float32) + hb2_ref[t][None, :]
        preds_ref[:, t:t + 1] = p


_moe_call = pl.pallas_call(
    _moe_body,
    out_shape=(jax.ShapeDtypeStruct((_M, _NT), jnp.float32),
               jax.ShapeDtypeStruct((1, 1), jnp.float32)),
)


def kernel(x, edge_index, batch, mol_idx, gin_W1, gin_b1, gin_W2, gin_b2,
           gin_eps, gate_W, exp_W1, exp_b1, exp_W2, exp_b2, exp_W3, exp_b3,
           head_W1, head_b1, head_W2, head_b2):
    idx5 = edge_index.astype(jnp.int32).reshape(2, _NC, _NS, _ECH, _EC)
    z625 = jnp.zeros((_NPT, _D), jnp.float32)
    h = x
    for i in range(3):
        agg = _make_edge_agg()(h, idx5, z625)
        h = _gin_call(gin_eps[i].reshape(1, 1), h, agg[0], agg[1],
                      gin_W1[i], gin_b1[i].reshape(1, _D),
                      gin_W2[i], gin_b2[i].reshape(1, _D))
    hp = jnp.pad(h, ((0, _NP - _N), (0, 0)))
    bidx = jnp.pad(batch.astype(jnp.int32), (0, _NP - _N)).reshape(_NC, _NS, _PCH, _PC)
    cval = jnp.pad(jnp.ones((_N, _CW), jnp.float32), ((0, _NP - _N), (0, 0)))
    zs = jnp.zeros((_FPT, _D), jnp.float32)
    zc = jnp.zeros((_FPT, _CW), jnp.float32)
    fsum, fcnt = _make_pool()(hp, bidx, cval, zs, zc)
    preds, lb = _moe_call(fsum, fcnt, gate_W, exp_W1, exp_b1, exp_W2, exp_b2,
                          exp_W3, exp_b3, mol_idx.astype(jnp.int32).reshape(_F, 1),
                          head_W1, head_b1, head_W2, head_b2)
    return preds, lb[0, 0]


# speedup vs baseline: 6.6305x; 1.1608x over previous
import functools

import jax
import jax.numpy as jnp
from jax import lax
from jax.experimental import pallas as pl
from jax.experimental.pallas import tpu as pltpu
from jax.experimental.pallas import tpu_sc as plsc

_N = 10000
_E = 320000
_F = 2048
_M = 256
_D = 128
_H = 256
_NE = 8
_NT = 3
_NC = 2
_NS = 16
_EC = 125
_ECH = 160
_DH = 64
_NP = 10240
_NPT = _NP // _NS
_PC = 64
_PCH = 5
_FPT = _F // _NS
_CW = 128
_BN_SCALE = 1.0 / (1.0 + 1e-5) ** 0.5
_PREC = lax.Precision.HIGHEST


def _edge_agg_body(hl_hbm, hr_hbm, idx_hbm, z_hbm, out_hbm, src_v, dst_v,
                   rows_a, rows_b, acc_sh, sem_a, sem_b):
    c = lax.axis_index("c")
    s = lax.axis_index("s")
    pltpu.sync_copy(z_hbm, acc_sh.at[pl.ds(s * _NPT, _NPT)])
    pltpu.sync_copy(idx_hbm.at[0, s], src_v)
    pltpu.sync_copy(idx_hbm.at[1, s], dst_v)
    plsc.subcore_barrier()

    def _run(h_hbm):
        pltpu.async_copy(h_hbm.at[src_v.at[0]], rows_a, sem_a)

        def _step(jj, carry):
            j0 = 2 * jj
            j1 = j0 + 1
            pltpu.async_copy(h_hbm.at[src_v.at[j1]], rows_b, sem_b)
            pltpu.make_async_copy(h_hbm.at[src_v.at[j0]], rows_a, sem_a).wait()
            pltpu.sync_copy(rows_a, acc_sh.at[dst_v.at[j0]], add=True)

            @pl.when(jj < _ECH // 2 - 1)
            def _():
                pltpu.async_copy(h_hbm.at[src_v.at[j0 + 2]], rows_a, sem_a)

            pltpu.make_async_copy(h_hbm.at[src_v.at[j1]], rows_b, sem_b).wait()
            pltpu.sync_copy(rows_b, acc_sh.at[dst_v.at[j1]], add=True)
            return carry

        lax.fori_loop(0, _ECH // 2, _step, 0)

    @pl.when(c == 0)
    def _():
        _run(hl_hbm)

    @pl.when(c == 1)
    def _():
        _run(hr_hbm)

    plsc.subcore_barrier()
    pltpu.sync_copy(acc_sh.at[pl.ds(s * _NPT, _NPT)],
                    out_hbm.at[c, pl.ds(s * _NPT, _NPT)])


@functools.lru_cache(maxsize=None)
def _make_edge_agg():
    return pl.kernel(
        _edge_agg_body,
        out_type=jax.ShapeDtypeStruct((_NC, _NP, _DH), jnp.float32),
        mesh=plsc.VectorSubcoreMesh(core_axis_name="c", subcore_axis_name="s"),
        compiler_params=pltpu.CompilerParams(use_tc_tiling_on_sc=False),
        scratch_types=[
            pltpu.VMEM((_ECH, _EC), jnp.int32),
            pltpu.VMEM((_ECH, _EC), jnp.int32),
            pltpu.VMEM((_EC, _DH), jnp.float32),
            pltpu.VMEM((_EC, _DH), jnp.float32),
            pltpu.VMEM_SHARED((_NP, _DH), jnp.float32),
            pltpu.SemaphoreType.DMA,
            pltpu.SemaphoreType.DMA,
        ],
    )


def _pool_body(h_hbm, bidx_hbm, cval_hbm, zs_hbm, zc_hbm, sum_hbm, cnt_hbm,
               idx_v, rows_v, cval_v, acc_sh, cacc_sh):
    c = lax.axis_index("c")
    s = lax.axis_index("s")
    w = c * _NS + s
    pltpu.sync_copy(zs_hbm, acc_sh.at[pl.ds(s * _FPT, _FPT)])
    pltpu.sync_copy(zc_hbm, cacc_sh.at[pl.ds(s * _FPT, _FPT)])
    pltpu.sync_copy(bidx_hbm.at[c, s], idx_v)
    plsc.subcore_barrier()
    base = w * (_PC * _PCH)

    def _step(j, carry):
        pltpu.sync_copy(h_hbm.at[pl.ds(base + j * _PC, _PC)], rows_v)
        pltpu.sync_copy(cval_hbm.at[pl.ds(base + j * _PC, _PC)], cval_v)
        pltpu.sync_copy(rows_v, acc_sh.at[idx_v.at[j]], add=True)
        pltpu.sync_copy(cval_v, cacc_sh.at[idx_v.at[j]], add=True)
        return carry

    lax.fori_loop(0, _PCH, _step, 0)
    plsc.subcore_barrier()
    pltpu.sync_copy(acc_sh.at[pl.ds(s * _FPT, _FPT)],
                    sum_hbm.at[c, pl.ds(s * _FPT, _FPT)])
    pltpu.sync_copy(cacc_sh.at[pl.ds(s * _FPT, _FPT)],
                    cnt_hbm.at[c, pl.ds(s * _FPT, _FPT)])


@functools.lru_cache(maxsize=None)
def _make_pool():
    return pl.kernel(
        _pool_body,
        out_type=(jax.ShapeDtypeStruct((_NC, _F, _D), jnp.float32),
                  jax.ShapeDtypeStruct((_NC, _F, _CW), jnp.float32)),
        mesh=plsc.VectorSubcoreMesh(core_axis_name="c", subcore_axis_name="s"),
        scratch_types=[
            pltpu.VMEM((_PCH, _PC), jnp.int32),
            pltpu.VMEM((_PC, _D), jnp.float32),
            pltpu.VMEM((_PC, _CW), jnp.float32),
            pltpu.VMEM_SHARED((_F, _D), jnp.float32),
            pltpu.VMEM_SHARED((_F, _CW), jnp.float32),
        ],
    )


def _gin_body(eps_ref, x_ref, al_ref, ar_ref, w1_ref, b1_ref, w2_ref, b2_ref,
              o_ref, ol_ref, or_ref):
    agg = jnp.concatenate([al_ref[...], ar_ref[...]], axis=1)
    m = (1.0 + eps_ref[0, 0]) * x_ref[...] + agg
    z = jnp.dot(m, w1_ref[...], precision=_PREC, preferred_element_type=jnp.float32)
    z = jnp.maximum(z + b1_ref[...], 0.0)
    z = jnp.dot(z, w2_ref[...], precision=_PREC, preferred_element_type=jnp.float32)
    z = (z + b2_ref[...]) * _BN_SCALE
    h = jnp.maximum(z, 0.0)
    o_ref[...] = h
    ol_ref[...] = h[:, :_DH]
    or_ref[...] = h[:, _DH:]


_GIN_BLK = 1000
_gin_call = pl.pallas_call(
    _gin_body,
    grid=(_N // _GIN_BLK,),
    in_specs=[
        pl.BlockSpec((1, 1), lambda i: (0, 0)),
        pl.BlockSpec((_GIN_BLK, _D), lambda i: (i, 0)),
        pl.BlockSpec((_GIN_BLK, _DH), lambda i: (i, 0)),
        pl.BlockSpec((_GIN_BLK, _DH), lambda i: (i, 0)),
        pl.BlockSpec((_D, _D), lambda i: (0, 0)),
        pl.BlockSpec((1, _D), lambda i: (0, 0)),
        pl.BlockSpec((_D, _D), lambda i: (0, 0)),
        pl.BlockSpec((1, _D), lambda i: (0, 0)),
    ],
    out_specs=(pl.BlockSpec((_GIN_BLK, _D), lambda i: (i, 0)),
               pl.BlockSpec((_GIN_BLK, _DH), lambda i: (i, 0)),
               pl.BlockSpec((_GIN_BLK, _DH), lambda i: (i, 0))),
    out_shape=(jax.ShapeDtypeStruct((_N, _D), jnp.float32),
               jax.ShapeDtypeStruct((_N, _DH), jnp.float32),
               jax.ShapeDtypeStruct((_N, _DH), jnp.float32)),
)


def _gelu(z):
    return 0.5 * z * (1.0 + lax.erf(z * 0.7071067811865476))


def _moe_body(sum_ref, cnt_ref, gate_ref, w1_ref, b1_ref, w2_ref, b2_ref,
              w3_ref, b3_ref, mol_ref, hw1_ref, hb1_ref, hw2_ref, hb2_ref,
              preds_ref, lb_ref):
    fs = sum_ref[0] + sum_ref[1]
    cnt = cnt_ref[0, :, 0:1] + cnt_ref[1, :, 0:1]
    femb = fs / jnp.maximum(cnt, 1.0)

    logits = jnp.dot(femb, gate_ref[...], precision=_PREC,
                     preferred_element_type=jnp.float32)
    iota = lax.broadcasted_iota(jnp.int32, (_F, _NE), 1)
    v1 = jnp.max(logits, axis=1, keepdims=True)
    i1 = jnp.min(jnp.where(logits == v1, iota, _NE), axis=1, keepdims=True)
    l2 = jnp.where(iota == i1, -jnp.inf, logits)
    v2 = jnp.max(l2, axis=1, keepdims=True)
    i2 = jnp.min(jnp.where(l2 == v2, iota, _NE), axis=1, keepdims=True)
    e2 = jnp.exp(v2 - v1)
    denom = 1.0 + e2
    wts = (jnp.where(iota == i1, 1.0, 0.0)
           + jnp.where(iota == i2, e2, 0.0)) / denom
    load = jnp.sum(wts, axis=0, keepdims=True) / float(_F)
    lb_ref[...] = jnp.reshape(float(_NE) * jnp.sum(load * load), (1, 1))

    moe = jnp.zeros((_F, _D), jnp.float32)
    for e in range(_NE):
        z = jnp.dot(femb, w1_ref[e], precision=_PREC,
                    preferred_element_type=jnp.float32) + b1_ref[e][None, :]
        z = _gelu(z)
        z = jnp.dot(z, w2_ref[e], precision=_PREC,
                    preferred_element_type=jnp.float32) + b2_ref[e][None, :]
        z = _gelu(z)
        z = jnp.dot(z, w3_ref[e], precision=_PREC,
                    preferred_element_type=jnp.float32) + b3_ref[e][None, :]
        moe = moe + wts[:, e:e + 1] * z

    iota_m = lax.broadcasted_iota(jnp.int32, (_F, _M), 1)
    sel = jnp.where(mol_ref[...] == iota_m, 1.0, 0.0)
    msum = lax.dot_general(sel, moe, (((0,), (0,)), ((), ())),
                           precision=_PREC, preferred_element_type=jnp.float32)
    ones_col = jnp.ones((_F, 1), jnp.float32)
    mcnt = lax.dot_general(sel, ones_col, (((0,), (0,)), ((), ())),
                           precision=_PREC, preferred_element_type=jnp.float32)
    memb = msum / jnp.maximum(mcnt, 1.0)

    for t in range(_NT):
        z = jnp.dot(memb, hw1_ref[t], precision=_PREC,
                    preferred_element_type=jnp.float32) + hb1_ref[t][None, :]
        z = jnp.maximum(z, 0.0)
        p = jnp.dot(z, hw2_ref[t], precision=_PREC,
                    preferred_element_type=jnp.float32) + hb2_ref[t][None, :]
        preds_ref[:, t:t + 1] = p


_moe_call = pl.pallas_call(
    _moe_body,
    out_shape=(jax.ShapeDtypeStruct((_M, _NT), jnp.float32),
               jax.ShapeDtypeStruct((1, 1), jnp.float32)),
)


def kernel(x, edge_index, batch, mol_idx, gin_W1, gin_b1, gin_W2, gin_b2,
           gin_eps, gate_W, exp_W1, exp_b1, exp_W2, exp_b2, exp_W3, exp_b3,
           head_W1, head_b1, head_W2, head_b2):
    idx4 = edge_index.astype(jnp.int32).reshape(2, _NS, _ECH, _EC)
    zh = jnp.zeros((_NPT, _DH), jnp.float32)
    h, hl, hr = x, x[:, :_DH], x[:, _DH:]
    for i in range(3):
        agg = _make_edge_agg()(hl, hr, idx4, zh)
        h, hl, hr = _gin_call(gin_eps[i].reshape(1, 1), h, agg[0], agg[1],
                              gin_W1[i], gin_b1[i].reshape(1, _D),
                              gin_W2[i], gin_b2[i].reshape(1, _D))
    hp = jnp.pad(h, ((0, _NP - _N), (0, 0)))
    bidx = jnp.pad(batch.astype(jnp.int32), (0, _NP - _N)).reshape(_NC, _NS, _PCH, _PC)
    cval = jnp.pad(jnp.ones((_N, _CW), jnp.float32), ((0, _NP - _N), (0, 0)))
    zs = jnp.zeros((_FPT, _D), jnp.float32)
    zc = jnp.zeros((_FPT, _CW), jnp.float32)
    fsum, fcnt = _make_pool()(hp, bidx, cval, zs, zc)
    preds, lb = _moe_call(fsum, fcnt, gate_W, exp_W1, exp_b1, exp_W2, exp_b2,
                          exp_W3, exp_b3, mol_idx.astype(jnp.int32).reshape(_F, 1),
                          head_W1, head_b1, head_W2, head_b2)
    return preds, lb[0, 0]


# speedup vs baseline: 6.9124x; 1.0425x over previous
import functools

import jax
import jax.numpy as jnp
from jax import lax
from jax.experimental import pallas as pl
from jax.experimental.pallas import tpu as pltpu
from jax.experimental.pallas import tpu_sc as plsc

_N = 10000
_E = 320000
_F = 2048
_M = 256
_D = 128
_H = 256
_NE = 8
_NT = 3
_NC = 2
_NS = 16
_EC = 125
_ECH = 160
_DH = 64
_NP = 10240
_NPT = _NP // _NS
_PC = 64
_PCH = 5
_FPT = _F // _NS
_CW = 128
_BN_SCALE = 1.0 / (1.0 + 1e-5) ** 0.5
_PREC = lax.Precision.HIGHEST


def _edge_agg_body(hl_hbm, hr_hbm, idx_hbm, z_hbm, out_hbm, src_v, dst_v,
                   rows_a, rows_b, acc_sh, sem_a, sem_b):
    c = lax.axis_index("c")
    s = lax.axis_index("s")
    pltpu.sync_copy(z_hbm, acc_sh.at[pl.ds(s * _NPT, _NPT)])
    pltpu.sync_copy(idx_hbm.at[0, s], src_v)
    pltpu.sync_copy(idx_hbm.at[1, s], dst_v)
    plsc.subcore_barrier()

    def _run(h_hbm):
        pltpu.async_copy(h_hbm.at[src_v.at[0]], rows_a, sem_a)

        def _step(jj, carry):
            j0 = 2 * jj
            j1 = j0 + 1
            pltpu.async_copy(h_hbm.at[src_v.at[j1]], rows_b, sem_b)
            pltpu.make_async_copy(h_hbm.at[src_v.at[j0]], rows_a, sem_a).wait()
            pltpu.sync_copy(rows_a, acc_sh.at[dst_v.at[j0]], add=True)

            @pl.when(jj < _ECH // 2 - 1)
            def _():
                pltpu.async_copy(h_hbm.at[src_v.at[j0 + 2]], rows_a, sem_a)

            pltpu.make_async_copy(h_hbm.at[src_v.at[j1]], rows_b, sem_b).wait()
            pltpu.sync_copy(rows_b, acc_sh.at[dst_v.at[j1]], add=True)
            return carry

        lax.fori_loop(0, _ECH // 2, _step, 0)

    @pl.when(c == 0)
    def _():
        _run(hl_hbm)

    @pl.when(c == 1)
    def _():
        _run(hr_hbm)

    plsc.subcore_barrier()
    pltpu.sync_copy(acc_sh.at[pl.ds(s * _NPT, _NPT)],
                    out_hbm.at[c, pl.ds(s * _NPT, _NPT)])


@functools.lru_cache(maxsize=None)
def _make_edge_agg():
    return pl.kernel(
        _edge_agg_body,
        out_type=jax.ShapeDtypeStruct((_NC, _NP, _DH), jnp.float32),
        mesh=plsc.VectorSubcoreMesh(core_axis_name="c", subcore_axis_name="s"),
        compiler_params=pltpu.CompilerParams(use_tc_tiling_on_sc=False),
        scratch_types=[
            pltpu.VMEM((_ECH, _EC), jnp.int32),
            pltpu.VMEM((_ECH, _EC), jnp.int32),
            pltpu.VMEM((_EC, _DH), jnp.float32),
            pltpu.VMEM((_EC, _DH), jnp.float32),
            pltpu.VMEM_SHARED((_NP, _DH), jnp.float32),
            pltpu.SemaphoreType.DMA,
            pltpu.SemaphoreType.DMA,
        ],
    )


def _pool_body(h_hbm, bidx_hbm, cval_hbm, zs_hbm, zc_hbm, sum_hbm, cnt_hbm,
               idx_v, rows_v, cval_v, acc_sh, cacc_sh):
    c = lax.axis_index("c")
    s = lax.axis_index("s")
    w = c * _NS + s
    pltpu.sync_copy(zs_hbm, acc_sh.at[pl.ds(s * _FPT, _FPT)])
    pltpu.sync_copy(zc_hbm, cacc_sh.at[pl.ds(s * _FPT, _FPT)])
    pltpu.sync_copy(bidx_hbm.at[c, s], idx_v)
    plsc.subcore_barrier()
    base = w * (_PC * _PCH)

    def _step(j, carry):
        pltpu.sync_copy(h_hbm.at[pl.ds(base + j * _PC, _PC)], rows_v)
        pltpu.sync_copy(cval_hbm.at[pl.ds(base + j * _PC, _PC)], cval_v)
        pltpu.sync_copy(rows_v, acc_sh.at[idx_v.at[j]], add=True)
        pltpu.sync_copy(cval_v, cacc_sh.at[idx_v.at[j]], add=True)
        return carry

    lax.fori_loop(0, _PCH, _step, 0)
    plsc.subcore_barrier()
    pltpu.sync_copy(acc_sh.at[pl.ds(s * _FPT, _FPT)],
                    sum_hbm.at[c, pl.ds(s * _FPT, _FPT)])
    pltpu.sync_copy(cacc_sh.at[pl.ds(s * _FPT, _FPT)],
                    cnt_hbm.at[c, pl.ds(s * _FPT, _FPT)])


@functools.lru_cache(maxsize=None)
def _make_pool():
    return pl.kernel(
        _pool_body,
        out_type=(jax.ShapeDtypeStruct((_NC, _F, _D), jnp.float32),
                  jax.ShapeDtypeStruct((_NC, _F, _CW), jnp.float32)),
        mesh=plsc.VectorSubcoreMesh(core_axis_name="c", subcore_axis_name="s"),
        scratch_types=[
            pltpu.VMEM((_PCH, _PC), jnp.int32),
            pltpu.VMEM((_PC, _D), jnp.float32),
            pltpu.VMEM((_PC, _CW), jnp.float32),
            pltpu.VMEM_SHARED((_F, _D), jnp.float32),
            pltpu.VMEM_SHARED((_F, _CW), jnp.float32),
        ],
    )


def _gin_body(eps_ref, x_ref, al_ref, ar_ref, w1_ref, b1_ref, w2_ref, b2_ref,
              o_ref, ol_ref, or_ref):
    agg = jnp.concatenate([al_ref[...], ar_ref[...]], axis=1)
    m = (1.0 + eps_ref[0, 0]) * x_ref[...] + agg
    z = jnp.dot(m, w1_ref[...], precision=_PREC, preferred_element_type=jnp.float32)
    z = jnp.maximum(z + b1_ref[...], 0.0)
    z = jnp.dot(z, w2_ref[...], precision=_PREC, preferred_element_type=jnp.float32)
    z = (z + b2_ref[...]) * _BN_SCALE
    h = jnp.maximum(z, 0.0)
    o_ref[...] = h
    ol_ref[...] = h[:, :_DH]
    or_ref[...] = h[:, _DH:]


_GIN_BLK = 1000
_gin_call = pl.pallas_call(
    _gin_body,
    grid=(_N // _GIN_BLK,),
    in_specs=[
        pl.BlockSpec((1, 1), lambda i: (0, 0)),
        pl.BlockSpec((_GIN_BLK, _D), lambda i: (i, 0)),
        pl.BlockSpec((_GIN_BLK, _DH), lambda i: (i, 0)),
        pl.BlockSpec((_GIN_BLK, _DH), lambda i: (i, 0)),
        pl.BlockSpec((_D, _D), lambda i: (0, 0)),
        pl.BlockSpec((1, _D), lambda i: (0, 0)),
        pl.BlockSpec((_D, _D), lambda i: (0, 0)),
        pl.BlockSpec((1, _D), lambda i: (0, 0)),
    ],
    out_specs=(pl.BlockSpec((_GIN_BLK, _D), lambda i: (i, 0)),
               pl.BlockSpec((_GIN_BLK, _DH), lambda i: (i, 0)),
               pl.BlockSpec((_GIN_BLK, _DH), lambda i: (i, 0))),
    out_shape=(jax.ShapeDtypeStruct((_N, _D), jnp.float32),
               jax.ShapeDtypeStruct((_N, _DH), jnp.float32),
               jax.ShapeDtypeStruct((_N, _DH), jnp.float32)),
)


def _dot3(a, b):
    a_hi = a.astype(jnp.bfloat16)
    a_lo = (a - a_hi.astype(jnp.float32)).astype(jnp.bfloat16)
    b_hi = b.astype(jnp.bfloat16)
    b_lo = (b - b_hi.astype(jnp.float32)).astype(jnp.bfloat16)

    def d(x, y):
        return jnp.dot(x, y, preferred_element_type=jnp.float32)

    return d(a_hi, b_hi) + d(a_hi, b_lo) + d(a_lo, b_hi)


def _gelu(z):
    return 0.5 * z * (1.0 + lax.erf(z * 0.7071067811865476))


def _moe_body(sum_ref, cnt_ref, gate_ref, w1_ref, b1_ref, w2_ref, b2_ref,
              w3_ref, b3_ref, mol_ref, hw1_ref, hb1_ref, hw2_ref, hb2_ref,
              preds_ref, lb_ref):
    fs = sum_ref[0] + sum_ref[1]
    cnt = cnt_ref[0, :, 0:1] + cnt_ref[1, :, 0:1]
    femb = fs / jnp.maximum(cnt, 1.0)

    logits = jnp.dot(femb, gate_ref[...], precision=_PREC,
                     preferred_element_type=jnp.float32)
    iota = lax.broadcasted_iota(jnp.int32, (_F, _NE), 1)
    v1 = jnp.max(logits, axis=1, keepdims=True)
    i1 = jnp.min(jnp.where(logits == v1, iota, _NE), axis=1, keepdims=True)
    l2 = jnp.where(iota == i1, -jnp.inf, logits)
    v2 = jnp.max(l2, axis=1, keepdims=True)
    i2 = jnp.min(jnp.where(l2 == v2, iota, _NE), axis=1, keepdims=True)
    e2 = jnp.exp(v2 - v1)
    denom = 1.0 + e2
    wts = (jnp.where(iota == i1, 1.0, 0.0)
           + jnp.where(iota == i2, e2, 0.0)) / denom
    load = jnp.sum(wts, axis=0, keepdims=True) / float(_F)
    lb_ref[...] = jnp.reshape(float(_NE) * jnp.sum(load * load), (1, 1))

    moe = jnp.zeros((_F, _D), jnp.float32)
    for e in range(_NE):
        z = _dot3(femb, w1_ref[e]) + b1_ref[e][None, :]
        z = _gelu(z)
        z = _dot3(z, w2_ref[e]) + b2_ref[e][None, :]
        z = _gelu(z)
        z = _dot3(z, w3_ref[e]) + b3_ref[e][None, :]
        moe = moe + wts[:, e:e + 1] * z

    iota_m = lax.broadcasted_iota(jnp.int32, (_F, _M), 1)
    sel = jnp.where(mol_ref[...] == iota_m, 1.0, 0.0)
    msum = lax.dot_general(sel, moe, (((0,), (0,)), ((), ())),
                           precision=_PREC, preferred_element_type=jnp.float32)
    ones_col = jnp.ones((_F, 1), jnp.float32)
    mcnt = lax.dot_general(sel, ones_col, (((0,), (0,)), ((), ())),
                           precision=_PREC, preferred_element_type=jnp.float32)
    memb = msum / jnp.maximum(mcnt, 1.0)

    for t in range(_NT):
        z = jnp.dot(memb, hw1_ref[t], precision=_PREC,
                    preferred_element_type=jnp.float32) + hb1_ref[t][None, :]
        z = jnp.maximum(z, 0.0)
        p = jnp.dot(z, hw2_ref[t], precision=_PREC,
                    preferred_element_type=jnp.float32) + hb2_ref[t][None, :]
        preds_ref[:, t:t + 1] = p


_moe_call = pl.pallas_call(
    _moe_body,
    out_shape=(jax.ShapeDtypeStruct((_M, _NT), jnp.float32),
               jax.ShapeDtypeStruct((1, 1), jnp.float32)),
)


def kernel(x, edge_index, batch, mol_idx, gin_W1, gin_b1, gin_W2, gin_b2,
           gin_eps, gate_W, exp_W1, exp_b1, exp_W2, exp_b2, exp_W3, exp_b3,
           head_W1, head_b1, head_W2, head_b2):
    idx4 = edge_index.astype(jnp.int32).reshape(2, _NS, _ECH, _EC)
    zh = jnp.zeros((_NPT, _DH), jnp.float32)
    h, hl, hr = x, x[:, :_DH], x[:, _DH:]
    for i in range(3):
        agg = _make_edge_agg()(hl, hr, idx4, zh)
        h, hl, hr = _gin_call(gin_eps[i].reshape(1, 1), h, agg[0], agg[1],
                              gin_W1[i], gin_b1[i].reshape(1, _D),
                              gin_W2[i], gin_b2[i].reshape(1, _D))
    hp = jnp.pad(h, ((0, _NP - _N), (0, 0)))
    bidx = jnp.pad(batch.astype(jnp.int32), (0, _NP - _N)).reshape(_NC, _NS, _PCH, _PC)
    cval = jnp.pad(jnp.ones((_N, _CW), jnp.float32), ((0, _NP - _N), (0, 0)))
    zs = jnp.zeros((_FPT, _D), jnp.float32)
    zc = jnp.zeros((_FPT, _CW), jnp.float32)
    fsum, fcnt = _make_pool()(hp, bidx, cval, zs, zc)
    preds, lb = _moe_call(fsum, fcnt, gate_W, exp_W1, exp_b1, exp_W2, exp_b2,
                          exp_W3, exp_b3, mol_idx.astype(jnp.int32).reshape(_F, 1),
                          head_W1, head_b1, head_W2, head_b2)
    return preds, lb[0, 0]


# speedup vs baseline: 8.8860x; 1.2855x over previous
import functools

import jax
import jax.numpy as jnp
from jax import lax
from jax.experimental import pallas as pl
from jax.experimental.pallas import tpu as pltpu
from jax.experimental.pallas import tpu_sc as plsc

_N = 10000
_E = 320000
_F = 2048
_M = 256
_D = 128
_H = 256
_NE = 8
_NT = 3
_NC = 2
_NS = 16
_EC = 125
_ECH = 160
_DH = 64
_NBUF = 4
_NP = 10240
_NPT = _NP // _NS
_PC = 64
_PCH = 5
_FPT = _F // _NS
_CW = 128
_BN_SCALE = 1.0 / (1.0 + 1e-5) ** 0.5
_PREC = lax.Precision.HIGHEST


def _edge_agg_body(hl_hbm, hr_hbm, idx_hbm, z_hbm, out_hbm, src_v, dst_v,
                   rows_a, rows_b, rows_c, rows_d, acc_sh,
                   sem_a, sem_b, sem_c, sem_d):
    c = lax.axis_index("c")
    s = lax.axis_index("s")
    pltpu.sync_copy(z_hbm, acc_sh.at[pl.ds(s * _NPT, _NPT)])
    pltpu.sync_copy(idx_hbm.at[0, s], src_v)
    pltpu.sync_copy(idx_hbm.at[1, s], dst_v)
    plsc.subcore_barrier()

    def _run(h_hbm):
        rows = (rows_a, rows_b, rows_c, rows_d)
        sems = (sem_a, sem_b, sem_c, sem_d)
        for b in range(_NBUF - 1):
            pltpu.async_copy(h_hbm.at[src_v.at[b]], rows[b], sems[b])

        def _step(jj, carry):
            base = _NBUF * jj
            for b in range(_NBUF):
                j = base + b
                nxt = j + (_NBUF - 1)
                nb = (b + _NBUF - 1) % _NBUF
                @pl.when(nxt < _ECH)
                def _():
                    pltpu.async_copy(h_hbm.at[src_v.at[nxt]], rows[nb], sems[nb])
                pltpu.make_async_copy(h_hbm.at[src_v.at[j]], rows[b], sems[b]).wait()
                pltpu.sync_copy(rows[b], acc_sh.at[dst_v.at[j]], add=True)
            return carry

        lax.fori_loop(0, _ECH // _NBUF, _step, 0)

    @pl.when(c == 0)
    def _():
        _run(hl_hbm)

    @pl.when(c == 1)
    def _():
        _run(hr_hbm)

    plsc.subcore_barrier()
    pltpu.sync_copy(acc_sh.at[pl.ds(s * _NPT, _NPT)],
                    out_hbm.at[c, pl.ds(s * _NPT, _NPT)])


@functools.lru_cache(maxsize=None)
def _make_edge_agg():
    return pl.kernel(
        _edge_agg_body,
        out_type=jax.ShapeDtypeStruct((_NC, _NP, _DH), jnp.float32),
        mesh=plsc.VectorSubcoreMesh(core_axis_name="c", subcore_axis_name="s"),
        compiler_params=pltpu.CompilerParams(use_tc_tiling_on_sc=False),
        scratch_types=[
            pltpu.VMEM((_ECH, _EC), jnp.int32),
            pltpu.VMEM((_ECH, _EC), jnp.int32),
            pltpu.VMEM((_EC, _DH), jnp.float32),
            pltpu.VMEM((_EC, _DH), jnp.float32),
            pltpu.VMEM((_EC, _DH), jnp.float32),
            pltpu.VMEM((_EC, _DH), jnp.float32),
            pltpu.VMEM_SHARED((_NP, _DH), jnp.float32),
            pltpu.SemaphoreType.DMA,
            pltpu.SemaphoreType.DMA,
            pltpu.SemaphoreType.DMA,
            pltpu.SemaphoreType.DMA,
        ],
    )


def _pool_body(h_hbm, bidx_hbm, cval_hbm, zs_hbm, zc_hbm, sum_hbm, cnt_hbm,
               idx_v, rows_v, cval_v, acc_sh, cacc_sh):
    c = lax.axis_index("c")
    s = lax.axis_index("s")
    w = c * _NS + s
    pltpu.sync_copy(zs_hbm, acc_sh.at[pl.ds(s * _FPT, _FPT)])
    pltpu.sync_copy(zc_hbm, cacc_sh.at[pl.ds(s * _FPT, _FPT)])
    pltpu.sync_copy(bidx_hbm.at[c, s], idx_v)
    plsc.subcore_barrier()
    base = w * (_PC * _PCH)

    def _step(j, carry):
        pltpu.sync_copy(h_hbm.at[pl.ds(base + j * _PC, _PC)], rows_v)
        pltpu.sync_copy(cval_hbm.at[pl.ds(base + j * _PC, _PC)], cval_v)
        pltpu.sync_copy(rows_v, acc_sh.at[idx_v.at[j]], add=True)
        pltpu.sync_copy(cval_v, cacc_sh.at[idx_v.at[j]], add=True)
        return carry

    lax.fori_loop(0, _PCH, _step, 0)
    plsc.subcore_barrier()
    pltpu.sync_copy(acc_sh.at[pl.ds(s * _FPT, _FPT)],
                    sum_hbm.at[c, pl.ds(s * _FPT, _FPT)])
    pltpu.sync_copy(cacc_sh.at[pl.ds(s * _FPT, _FPT)],
                    cnt_hbm.at[c, pl.ds(s * _FPT, _FPT)])


@functools.lru_cache(maxsize=None)
def _make_pool():
    return pl.kernel(
        _pool_body,
        out_type=(jax.ShapeDtypeStruct((_NC, _F, _D), jnp.float32),
                  jax.ShapeDtypeStruct((_NC, _F, _CW), jnp.float32)),
        mesh=plsc.VectorSubcoreMesh(core_axis_name="c", subcore_axis_name="s"),
        scratch_types=[
            pltpu.VMEM((_PCH, _PC), jnp.int32),
            pltpu.VMEM((_PC, _D), jnp.float32),
            pltpu.VMEM((_PC, _CW), jnp.float32),
            pltpu.VMEM_SHARED((_F, _D), jnp.float32),
            pltpu.VMEM_SHARED((_F, _CW), jnp.float32),
        ],
    )


def _dot3(a, b):
    a_hi = a.astype(jnp.bfloat16)
    a_lo = (a - a_hi.astype(jnp.float32)).astype(jnp.bfloat16)
    b_hi = b.astype(jnp.bfloat16)
    b_lo = (b - b_hi.astype(jnp.float32)).astype(jnp.bfloat16)

    def d(x, y):
        return jnp.dot(x, y, preferred_element_type=jnp.float32)

    return d(a_hi, b_hi) + d(a_hi, b_lo) + d(a_lo, b_hi)


def _gin_body(eps_ref, x_ref, al_ref, ar_ref, w1_ref, b1_ref, w2_ref, b2_ref,
              o_ref, ol_ref, or_ref):
    agg = jnp.concatenate([al_ref[...], ar_ref[...]], axis=1)
    m = (1.0 + eps_ref[0, 0]) * x_ref[...] + agg
    z = _dot3(m, w1_ref[...])
    z = jnp.maximum(z + b1_ref[...], 0.0)
    z = _dot3(z, w2_ref[...])
    z = (z + b2_ref[...]) * _BN_SCALE
    h = jnp.maximum(z, 0.0)
    o_ref[...] = h
    ol_ref[...] = h[:, :_DH]
    or_ref[...] = h[:, _DH:]


_GIN_BLK = 1000
_gin_call = pl.pallas_call(
    _gin_body,
    grid=(_N // _GIN_BLK,),
    in_specs=[
        pl.BlockSpec((1, 1), lambda i: (0, 0)),
        pl.BlockSpec((_GIN_BLK, _D), lambda i: (i, 0)),
        pl.BlockSpec((_GIN_BLK, _DH), lambda i: (i, 0)),
        pl.BlockSpec((_GIN_BLK, _DH), lambda i: (i, 0)),
        pl.BlockSpec((_D, _D), lambda i: (0, 0)),
        pl.BlockSpec((1, _D), lambda i: (0, 0)),
        pl.BlockSpec((_D, _D), lambda i: (0, 0)),
        pl.BlockSpec((1, _D), lambda i: (0, 0)),
    ],
    out_specs=(pl.BlockSpec((_GIN_BLK, _D), lambda i: (i, 0)),
               pl.BlockSpec((_GIN_BLK, _DH), lambda i: (i, 0)),
               pl.BlockSpec((_GIN_BLK, _DH), lambda i: (i, 0))),
    out_shape=(jax.ShapeDtypeStruct((_N, _D), jnp.float32),
               jax.ShapeDtypeStruct((_N, _DH), jnp.float32),
               jax.ShapeDtypeStruct((_N, _DH), jnp.float32)),
)


def _gelu(z):
    return 0.5 * z * (1.0 + lax.erf(z * 0.7071067811865476))


def _moe_body(sum_ref, cnt_ref, gate_ref, w1_ref, b1_ref, w2_ref, b2_ref,
              w3_ref, b3_ref, mol_ref, hw1_ref, hb1_ref, hw2_ref, hb2_ref,
              preds_ref, lb_ref):
    fs = sum_ref[0] + sum_ref[1]
    cnt = cnt_ref[0, :, 0:1] + cnt_ref[1, :, 0:1]
    femb = fs / jnp.maximum(cnt, 1.0)

    logits = jnp.dot(femb, gate_ref[...], precision=_PREC,
                     preferred_element_type=jnp.float32)
    iota = lax.broadcasted_iota(jnp.int32, (_F, _NE), 1)
    v1 = jnp.max(logits, axis=1, keepdims=True)
    i1 = jnp.min(jnp.where(logits == v1, iota, _NE), axis=1, keepdims=True)
    l2 = jnp.where(iota == i1, -jnp.inf, logits)
    v2 = jnp.max(l2, axis=1, keepdims=True)
    i2 = jnp.min(jnp.where(l2 == v2, iota, _NE), axis=1, keepdims=True)
    e2 = jnp.exp(v2 - v1)
    denom = 1.0 + e2
    wts = (jnp.where(iota == i1, 1.0, 0.0)
           + jnp.where(iota == i2, e2, 0.0)) / denom
    load = jnp.sum(wts, axis=0, keepdims=True) / float(_F)
    lb_ref[...] = jnp.reshape(float(_NE) * jnp.sum(load * load), (1, 1))

    moe = jnp.zeros((_F, _D), jnp.float32)
    for e in range(_NE):
        z = _dot3(femb, w1_ref[e]) + b1_ref[e][None, :]
        z = _gelu(z)
        z = _dot3(z, w2_ref[e]) + b2_ref[e][None, :]
        z = _gelu(z)
        z = _dot3(z, w3_ref[e]) + b3_ref[e][None, :]
        moe = moe + wts[:, e:e + 1] * z

    iota_m = lax.broadcasted_iota(jnp.int32, (_F, _M), 1)
    sel = jnp.where(mol_ref[...] == iota_m, 1.0, 0.0)
    msum = lax.dot_general(sel, moe, (((0,), (0,)), ((), ())),
                           precision=_PREC, preferred_element_type=jnp.float32)
    ones_col = jnp.ones((_F, 1), jnp.float32)
    mcnt = lax.dot_general(sel, ones_col, (((0,), (0,)), ((), ())),
                           precision=_PREC, preferred_element_type=jnp.float32)
    memb = msum / jnp.maximum(mcnt, 1.0)

    for t in range(_NT):
        z = jnp.dot(memb, hw1_ref[t], precision=_PREC,
                    preferred_element_type=jnp.float32) + hb1_ref[t][None, :]
        z = jnp.maximum(z, 0.0)
        p = jnp.dot(z, hw2_ref[t], precision=_PREC,
                    preferred_element_type=jnp.float32) + hb2_ref[t][None, :]
        preds_ref[:, t:t + 1] = p


_moe_call = pl.pallas_call(
    _moe_body,
    out_shape=(jax.ShapeDtypeStruct((_M, _NT), jnp.float32),
               jax.ShapeDtypeStruct((1, 1), jnp.float32)),
)


def kernel(x, edge_index, batch, mol_idx, gin_W1, gin_b1, gin_W2, gin_b2,
           gin_eps, gate_W, exp_W1, exp_b1, exp_W2, exp_b2, exp_W3, exp_b3,
           head_W1, head_b1, head_W2, head_b2):
    idx4 = edge_index.astype(jnp.int32).reshape(2, _NS, _ECH, _EC)
    zh = jnp.zeros((_NPT, _DH), jnp.float32)
    h, hl, hr = x, x[:, :_DH], x[:, _DH:]
    for i in range(3):
        agg = _make_edge_agg()(hl, hr, idx4, zh)
        h, hl, hr = _gin_call(gin_eps[i].reshape(1, 1), h, agg[0], agg[1],
                              gin_W1[i], gin_b1[i].reshape(1, _D),
                              gin_W2[i], gin_b2[i].reshape(1, _D))
    hp = jnp.pad(h, ((0, _NP - _N), (0, 0)))
    bidx = jnp.pad(batch.astype(jnp.int32), (0, _NP - _N)).reshape(_NC, _NS, _PCH, _PC)
    cval = jnp.pad(jnp.ones((_N, _CW), jnp.float32), ((0, _NP - _N), (0, 0)))
    zs = jnp.zeros((_FPT, _D), jnp.float32)
    zc = jnp.zeros((_FPT, _CW), jnp.float32)
    fsum, fcnt = _make_pool()(hp, bidx, cval, zs, zc)
    preds, lb = _moe_call(fsum, fcnt, gate_W, exp_W1, exp_b1, exp_W2, exp_b2,
                          exp_W3, exp_b3, mol_idx.astype(jnp.int32).reshape(_F, 1),
                          head_W1, head_b1, head_W2, head_b2)
    return preds, lb[0, 0]


# speedup vs baseline: 9.0275x; 1.0159x over previous
import functools

import jax
import jax.numpy as jnp
from jax import lax
from jax.experimental import pallas as pl
from jax.experimental.pallas import tpu as pltpu
from jax.experimental.pallas import tpu_sc as plsc

_N = 10000
_E = 320000
_F = 2048
_M = 256
_D = 128
_H = 256
_NE = 8
_NT = 3
_NC = 2
_NS = 16
_EC = 125
_ECH = 160
_DH = 64
_NBUF = 4
_NP = 10240
_NPT = _NP // _NS
_PC = 64
_PCH = 10
_FPT = _F // _NS
_FA = _F + 128
_FAT = _FA // _NS
_CW = 128
_BN_SCALE = 1.0 / (1.0 + 1e-5) ** 0.5
_PREC = lax.Precision.HIGHEST


def _edge_agg_body(hl_hbm, hr_hbm, idx_hbm, z_hbm, out_hbm, src_v, dst_v,
                   rows_a, rows_b, rows_c, rows_d, acc_sh,
                   sem_a, sem_b, sem_c, sem_d):
    c = lax.axis_index("c")
    s = lax.axis_index("s")
    pltpu.sync_copy(z_hbm, acc_sh.at[pl.ds(s * _NPT, _NPT)])
    pltpu.sync_copy(idx_hbm.at[0, s], src_v)
    pltpu.sync_copy(idx_hbm.at[1, s], dst_v)
    plsc.subcore_barrier()

    def _run(h_hbm):
        rows = (rows_a, rows_b, rows_c, rows_d)
        sems = (sem_a, sem_b, sem_c, sem_d)
        for b in range(_NBUF - 1):
            pltpu.async_copy(h_hbm.at[src_v.at[b]], rows[b], sems[b])

        def _step(jj, carry):
            base = _NBUF * jj
            for b in range(_NBUF):
                j = base + b
                nxt = j + (_NBUF - 1)
                nb = (b + _NBUF - 1) % _NBUF
                @pl.when(nxt < _ECH)
                def _():
                    pltpu.async_copy(h_hbm.at[src_v.at[nxt]], rows[nb], sems[nb])
                pltpu.make_async_copy(h_hbm.at[src_v.at[j]], rows[b], sems[b]).wait()
                pltpu.sync_copy(rows[b], acc_sh.at[dst_v.at[j]], add=True)
            return carry

        lax.fori_loop(0, _ECH // _NBUF, _step, 0)

    @pl.when(c == 0)
    def _():
        _run(hl_hbm)

    @pl.when(c == 1)
    def _():
        _run(hr_hbm)

    plsc.subcore_barrier()
    pltpu.sync_copy(acc_sh.at[pl.ds(s * _NPT, _NPT)],
                    out_hbm.at[c, pl.ds(s * _NPT, _NPT)])


@functools.lru_cache(maxsize=None)
def _make_edge_agg():
    return pl.kernel(
        _edge_agg_body,
        out_type=jax.ShapeDtypeStruct((_NC, _NP, _DH), jnp.float32),
        mesh=plsc.VectorSubcoreMesh(core_axis_name="c", subcore_axis_name="s"),
        compiler_params=pltpu.CompilerParams(use_tc_tiling_on_sc=False),
        scratch_types=[
            pltpu.VMEM((_ECH, _EC), jnp.int32),
            pltpu.VMEM((_ECH, _EC), jnp.int32),
            pltpu.VMEM((_EC, _DH), jnp.float32),
            pltpu.VMEM((_EC, _DH), jnp.float32),
            pltpu.VMEM((_EC, _DH), jnp.float32),
            pltpu.VMEM((_EC, _DH), jnp.float32),
            pltpu.VMEM_SHARED((_NP, _DH), jnp.float32),
            pltpu.SemaphoreType.DMA,
            pltpu.SemaphoreType.DMA,
            pltpu.SemaphoreType.DMA,
            pltpu.SemaphoreType.DMA,
        ],
    )


def _pool_body(h_hbm, bidx_hbm, ones_hbm, zs_hbm, sum_hbm, cnt_hbm,
               idx_v, rows_v, ones_v, acc_sh):
    c = lax.axis_index("c")
    s = lax.axis_index("s")
    pltpu.sync_copy(zs_hbm, acc_sh.at[pl.ds(s * _FAT, _FAT)])
    pltpu.sync_copy(bidx_hbm.at[s], idx_v)
    plsc.subcore_barrier()
    base = s * (_PC * _PCH)

    @pl.when(c == 0)
    def _():
        def _step(j, carry):
            pltpu.sync_copy(h_hbm.at[pl.ds(base + j * _PC, _PC)], rows_v)
            pltpu.sync_copy(rows_v, acc_sh.at[idx_v.at[j]], add=True)
            return carry
        lax.fori_loop(0, _PCH, _step, 0)

    @pl.when(c == 1)
    def _():
        pltpu.sync_copy(ones_hbm, ones_v)
        def _step(j, carry):
            pltpu.sync_copy(ones_v, acc_sh.at[idx_v.at[j]], add=True)
            return carry
        lax.fori_loop(0, _PCH, _step, 0)

    plsc.subcore_barrier()

    @pl.when(c == 0)
    def _():
        pltpu.sync_copy(acc_sh.at[pl.ds(s * _FPT, _FPT)],
                        sum_hbm.at[pl.ds(s * _FPT, _FPT)])

    @pl.when(c == 1)
    def _():
        pltpu.sync_copy(acc_sh.at[pl.ds(s * _FPT, _FPT)],
                        cnt_hbm.at[pl.ds(s * _FPT, _FPT)])


@functools.lru_cache(maxsize=None)
def _make_pool():
    return pl.kernel(
        _pool_body,
        out_type=(jax.ShapeDtypeStruct((_F, _D), jnp.float32),
                  jax.ShapeDtypeStruct((_F, _CW), jnp.float32)),
        mesh=plsc.VectorSubcoreMesh(core_axis_name="c", subcore_axis_name="s"),
        compiler_params=pltpu.CompilerParams(use_tc_tiling_on_sc=False),
        scratch_types=[
            pltpu.VMEM((_PCH, _PC), jnp.int32),
            pltpu.VMEM((_PC, _D), jnp.float32),
            pltpu.VMEM((_PC, _CW), jnp.float32),
            pltpu.VMEM_SHARED((_FA, _D), jnp.float32),
        ],
    )


def _dot3_pre(a_hi, a_lo, b):
    b_hi = b.astype(jnp.bfloat16)
    b_lo = (b - b_hi.astype(jnp.float32)).astype(jnp.bfloat16)

    def d(x, y):
        return jnp.dot(x, y, preferred_element_type=jnp.float32)

    return d(a_hi, b_hi) + d(a_hi, b_lo) + d(a_lo, b_hi)


def _dot3(a, b):
    a_hi = a.astype(jnp.bfloat16)
    a_lo = (a - a_hi.astype(jnp.float32)).astype(jnp.bfloat16)
    b_hi = b.astype(jnp.bfloat16)
    b_lo = (b - b_hi.astype(jnp.float32)).astype(jnp.bfloat16)

    def d(x, y):
        return jnp.dot(x, y, preferred_element_type=jnp.float32)

    return d(a_hi, b_hi) + d(a_hi, b_lo) + d(a_lo, b_hi)


def _gin_body(eps_ref, x_ref, al_ref, ar_ref, w1_ref, b1_ref, w2_ref, b2_ref,
              o_ref, ol_ref, or_ref):
    agg = jnp.concatenate([al_ref[...], ar_ref[...]], axis=1)
    m = (1.0 + eps_ref[0, 0]) * x_ref[...] + agg
    z = _dot3(m, w1_ref[...])
    z = jnp.maximum(z + b1_ref[...], 0.0)
    z = _dot3(z, w2_ref[...])
    z = (z + b2_ref[...]) * _BN_SCALE
    h = jnp.maximum(z, 0.0)
    o_ref[...] = h
    ol_ref[...] = h[:, :_DH]
    or_ref[...] = h[:, _DH:]


_GIN_BLK = 1000
_gin_call = pl.pallas_call(
    _gin_body,
    grid=(_N // _GIN_BLK,),
    in_specs=[
        pl.BlockSpec((1, 1), lambda i: (0, 0)),
        pl.BlockSpec((_GIN_BLK, _D), lambda i: (i, 0)),
        pl.BlockSpec((_GIN_BLK, _DH), lambda i: (i, 0)),
        pl.BlockSpec((_GIN_BLK, _DH), lambda i: (i, 0)),
        pl.BlockSpec((_D, _D), lambda i: (0, 0)),
        pl.BlockSpec((1, _D), lambda i: (0, 0)),
        pl.BlockSpec((_D, _D), lambda i: (0, 0)),
        pl.BlockSpec((1, _D), lambda i: (0, 0)),
    ],
    out_specs=(pl.BlockSpec((_GIN_BLK, _D), lambda i: (i, 0)),
               pl.BlockSpec((_GIN_BLK, _DH), lambda i: (i, 0)),
               pl.BlockSpec((_GIN_BLK, _DH), lambda i: (i, 0))),
    out_shape=(jax.ShapeDtypeStruct((_N, _D), jnp.float32),
               jax.ShapeDtypeStruct((_N, _DH), jnp.float32),
               jax.ShapeDtypeStruct((_N, _DH), jnp.float32)),
)


def _gelu(z):
    return 0.5 * z * (1.0 + lax.erf(z * 0.7071067811865476))


def _moe_body(sum_ref, cnt_ref, gate_ref, w1_ref, b1_ref, w2_ref, b2_ref,
              w3_ref, b3_ref, mol_ref, hw1_ref, hb1_ref, hw2_ref, hb2_ref,
              preds_ref, lb_ref):
    fs = sum_ref[...]
    cnt = cnt_ref[:, 0:1]
    femb = fs / jnp.maximum(cnt, 1.0)

    logits = jnp.dot(femb, gate_ref[...], precision=_PREC,
                     preferred_element_type=jnp.float32)
    iota = lax.broadcasted_iota(jnp.int32, (_F, _NE), 1)
    v1 = jnp.max(logits, axis=1, keepdims=True)
    i1 = jnp.min(jnp.where(logits == v1, iota, _NE), axis=1, keepdims=True)
    l2 = jnp.where(iota == i1, -jnp.inf, logits)
    v2 = jnp.max(l2, axis=1, keepdims=True)
    i2 = jnp.min(jnp.where(l2 == v2, iota, _NE), axis=1, keepdims=True)
    e2 = jnp.exp(v2 - v1)
    denom = 1.0 + e2
    wts = (jnp.where(iota == i1, 1.0, 0.0)
           + jnp.where(iota == i2, e2, 0.0)) / denom
    load = jnp.sum(wts, axis=0, keepdims=True) / float(_F)
    lb_ref[...] = jnp.reshape(float(_NE) * jnp.sum(load * load), (1, 1))

    moe = jnp.zeros((_F, _D), jnp.float32)
    femb_hi = femb.astype(jnp.bfloat16)
    femb_lo = (femb - femb_hi.astype(jnp.float32)).astype(jnp.bfloat16)
    for e in range(_NE):
        z = _dot3_pre(femb_hi, femb_lo, w1_ref[e]) + b1_ref[e][None, :]
        z = _gelu(z)
        z = _dot3(z, w2_ref[e]) + b2_ref[e][None, :]
        z = _gelu(z)
        z = _dot3(z, w3_ref[e]) + b3_ref[e][None, :]
        moe = moe + wts[:, e:e + 1] * z

    iota_m = lax.broadcasted_iota(jnp.int32, (_F, _M), 1)
    sel = jnp.where(mol_ref[...] == iota_m, 1.0, 0.0)
    msum = lax.dot_general(sel, moe, (((0,), (0,)), ((), ())),
                           precision=_PREC, preferred_element_type=jnp.float32)
    ones_col = jnp.ones((_F, 1), jnp.float32)
    mcnt = lax.dot_general(sel, ones_col, (((0,), (0,)), ((), ())),
                           precision=_PREC, preferred_element_type=jnp.float32)
    memb = msum / jnp.maximum(mcnt, 1.0)

    for t in range(_NT):
        z = jnp.dot(memb, hw1_ref[t], precision=_PREC,
                    preferred_element_type=jnp.float32) + hb1_ref[t][None, :]
        z = jnp.maximum(z, 0.0)
        p = jnp.dot(z, hw2_ref[t], precision=_PREC,
                    preferred_element_type=jnp.float32) + hb2_ref[t][None, :]
        preds_ref[:, t:t + 1] = p


_moe_call = pl.pallas_call(
    _moe_body,
    out_shape=(jax.ShapeDtypeStruct((_M, _NT), jnp.float32),
               jax.ShapeDtypeStruct((1, 1), jnp.float32)),
)


def kernel(x, edge_index, batch, mol_idx, gin_W1, gin_b1, gin_W2, gin_b2,
           gin_eps, gate_W, exp_W1, exp_b1, exp_W2, exp_b2, exp_W3, exp_b3,
           head_W1, head_b1, head_W2, head_b2):
    idx4 = edge_index.astype(jnp.int32).reshape(2, _NS, _ECH, _EC)
    zh = jnp.zeros((_NPT, _DH), jnp.float32)
    h, hl, hr = x, x[:, :_DH], x[:, _DH:]
    for i in range(3):
        agg = _make_edge_agg()(hl, hr, idx4, zh)
        h, hl, hr = _gin_call(gin_eps[i].reshape(1, 1), h, agg[0], agg[1],
                              gin_W1[i], gin_b1[i].reshape(1, _D),
                              gin_W2[i], gin_b2[i].reshape(1, _D))
    hp = jnp.pad(h, ((0, _NP - _N), (0, 0)))
    bidx = jnp.pad(batch.astype(jnp.int32), (0, _NP - _N),
                   constant_values=_F).reshape(_NS, _PCH, _PC)
    onesv = jnp.ones((_PC, _CW), jnp.float32)
    zs = jnp.zeros((_FAT, _D), jnp.float32)
    fsum, fcnt = _make_pool()(hp, bidx, onesv, zs)
    preds, lb = _moe_call(fsum, fcnt, gate_W, exp_W1, exp_b1, exp_W2, exp_b2,
                          exp_W3, exp_b3, mol_idx.astype(jnp.int32).reshape(_F, 1),
                          head_W1, head_b1, head_W2, head_b2)
    return preds, lb[0, 0]


# speedup vs baseline: 9.6022x; 1.0637x over previous
import functools

import jax
import jax.numpy as jnp
from jax import lax
from jax.experimental import pallas as pl
from jax.experimental.pallas import tpu as pltpu
from jax.experimental.pallas import tpu_sc as plsc

_N = 10000
_E = 320000
_F = 2048
_M = 256
_D = 128
_H = 256
_NE = 8
_NT = 3
_NC = 2
_NS = 16
_EC = 125
_ECH = 160
_DH = 64
_NBUF = 4
_NP = 10240
_NPT = _NP // _NS
_PC = 64
_PCH = 10
_FPT = _F // _NS
_FA = _F + 128
_FAT = _FA // _NS
_CW = 128
_BN_SCALE = 1.0 / (1.0 + 1e-5) ** 0.5
_PREC = lax.Precision.HIGHEST
_PREC_REF = lax.Precision.DEFAULT


def _edge_agg_body(hl_hbm, hr_hbm, idx_hbm, z_hbm, out_hbm, src_v, dst_v,
                   rows_a, rows_b, rows_c, rows_d, acc_sh,
                   sem_a, sem_b, sem_c, sem_d):
    c = lax.axis_index("c")
    s = lax.axis_index("s")
    pltpu.sync_copy(z_hbm, acc_sh.at[pl.ds(s * _NPT, _NPT)])
    pltpu.sync_copy(idx_hbm.at[0, s], src_v)
    pltpu.sync_copy(idx_hbm.at[1, s], dst_v)
    plsc.subcore_barrier()

    def _run(h_hbm):
        rows = (rows_a, rows_b, rows_c, rows_d)
        sems = (sem_a, sem_b, sem_c, sem_d)
        for b in range(_NBUF - 1):
            pltpu.async_copy(h_hbm.at[src_v.at[b]], rows[b], sems[b])

        def _step(jj, carry):
            base = _NBUF * jj
            for b in range(_NBUF):
                j = base + b
                nxt = j + (_NBUF - 1)
                nb = (b + _NBUF - 1) % _NBUF
                @pl.when(nxt < _ECH)
                def _():
                    pltpu.async_copy(h_hbm.at[src_v.at[nxt]], rows[nb], sems[nb])
                pltpu.make_async_copy(h_hbm.at[src_v.at[j]], rows[b], sems[b]).wait()
                pltpu.sync_copy(rows[b], acc_sh.at[dst_v.at[j]], add=True)
            return carry

        lax.fori_loop(0, _ECH // _NBUF, _step, 0)

    @pl.when(c == 0)
    def _():
        _run(hl_hbm)

    @pl.when(c == 1)
    def _():
        _run(hr_hbm)

    plsc.subcore_barrier()
    pltpu.sync_copy(acc_sh.at[pl.ds(s * _NPT, _NPT)],
                    out_hbm.at[c, pl.ds(s * _NPT, _NPT)])


@functools.lru_cache(maxsize=None)
def _make_edge_agg():
    return pl.kernel(
        _edge_agg_body,
        out_type=jax.ShapeDtypeStruct((_NC, _NP, _DH), jnp.float32),
        mesh=plsc.VectorSubcoreMesh(core_axis_name="c", subcore_axis_name="s"),
        compiler_params=pltpu.CompilerParams(use_tc_tiling_on_sc=False),
        scratch_types=[
            pltpu.VMEM((_ECH, _EC), jnp.int32),
            pltpu.VMEM((_ECH, _EC), jnp.int32),
            pltpu.VMEM((_EC, _DH), jnp.float32),
            pltpu.VMEM((_EC, _DH), jnp.float32),
            pltpu.VMEM((_EC, _DH), jnp.float32),
            pltpu.VMEM((_EC, _DH), jnp.float32),
            pltpu.VMEM_SHARED((_NP, _DH), jnp.float32),
            pltpu.SemaphoreType.DMA,
            pltpu.SemaphoreType.DMA,
            pltpu.SemaphoreType.DMA,
            pltpu.SemaphoreType.DMA,
        ],
    )


def _pool_body(h_hbm, bidx_hbm, ones_hbm, zs_hbm, sum_hbm, cnt_hbm,
               idx_v, rows_v, ones_v, acc_sh):
    c = lax.axis_index("c")
    s = lax.axis_index("s")
    pltpu.sync_copy(zs_hbm, acc_sh.at[pl.ds(s * _FAT, _FAT)])
    pltpu.sync_copy(bidx_hbm.at[s], idx_v)
    plsc.subcore_barrier()
    base = s * (_PC * _PCH)

    @pl.when(c == 0)
    def _():
        def _step(j, carry):
            pltpu.sync_copy(h_hbm.at[pl.ds(base + j * _PC, _PC)], rows_v)
            pltpu.sync_copy(rows_v, acc_sh.at[idx_v.at[j]], add=True)
            return carry
        lax.fori_loop(0, _PCH, _step, 0)

    @pl.when(c == 1)
    def _():
        pltpu.sync_copy(ones_hbm, ones_v)
        def _step(j, carry):
            pltpu.sync_copy(ones_v, acc_sh.at[idx_v.at[j]], add=True)
            return carry
        lax.fori_loop(0, _PCH, _step, 0)

    plsc.subcore_barrier()

    @pl.when(c == 0)
    def _():
        pltpu.sync_copy(acc_sh.at[pl.ds(s * _FPT, _FPT)],
                        sum_hbm.at[pl.ds(s * _FPT, _FPT)])

    @pl.when(c == 1)
    def _():
        pltpu.sync_copy(acc_sh.at[pl.ds(s * _FPT, _FPT)],
                        cnt_hbm.at[pl.ds(s * _FPT, _FPT)])


@functools.lru_cache(maxsize=None)
def _make_pool():
    return pl.kernel(
        _pool_body,
        out_type=(jax.ShapeDtypeStruct((_F, _D), jnp.float32),
                  jax.ShapeDtypeStruct((_F, _CW), jnp.float32)),
        mesh=plsc.VectorSubcoreMesh(core_axis_name="c", subcore_axis_name="s"),
        compiler_params=pltpu.CompilerParams(use_tc_tiling_on_sc=False),
        scratch_types=[
            pltpu.VMEM((_PCH, _PC), jnp.int32),
            pltpu.VMEM((_PC, _D), jnp.float32),
            pltpu.VMEM((_PC, _CW), jnp.float32),
            pltpu.VMEM_SHARED((_FA, _D), jnp.float32),
        ],
    )


def _gin_body(eps_ref, x_ref, al_ref, ar_ref, w1_ref, b1_ref, w2_ref, b2_ref,
              o_ref, ol_ref, or_ref):
    agg = jnp.concatenate([al_ref[...], ar_ref[...]], axis=1)
    m = (1.0 + eps_ref[0, 0]) * x_ref[...] + agg
    z = jnp.dot(m, w1_ref[...], precision=_PREC_REF, preferred_element_type=jnp.float32)
    z = jnp.maximum(z + b1_ref[...], 0.0)
    z = jnp.dot(z, w2_ref[...], precision=_PREC_REF, preferred_element_type=jnp.float32)
    z = (z + b2_ref[...]) * _BN_SCALE
    h = jnp.maximum(z, 0.0)
    o_ref[...] = h
    ol_ref[...] = h[:, :_DH]
    or_ref[...] = h[:, _DH:]


_GIN_BLK = 1000
_gin_call = pl.pallas_call(
    _gin_body,
    grid=(_N // _GIN_BLK,),
    in_specs=[
        pl.BlockSpec((1, 1), lambda i: (0, 0)),
        pl.BlockSpec((_GIN_BLK, _D), lambda i: (i, 0)),
        pl.BlockSpec((_GIN_BLK, _DH), lambda i: (i, 0)),
        pl.BlockSpec((_GIN_BLK, _DH), lambda i: (i, 0)),
        pl.BlockSpec((_D, _D), lambda i: (0, 0)),
        pl.BlockSpec((1, _D), lambda i: (0, 0)),
        pl.BlockSpec((_D, _D), lambda i: (0, 0)),
        pl.BlockSpec((1, _D), lambda i: (0, 0)),
    ],
    out_specs=(pl.BlockSpec((_GIN_BLK, _D), lambda i: (i, 0)),
               pl.BlockSpec((_GIN_BLK, _DH), lambda i: (i, 0)),
               pl.BlockSpec((_GIN_BLK, _DH), lambda i: (i, 0))),
    out_shape=(jax.ShapeDtypeStruct((_N, _D), jnp.float32),
               jax.ShapeDtypeStruct((_N, _DH), jnp.float32),
               jax.ShapeDtypeStruct((_N, _DH), jnp.float32)),
)


def _gelu(z):
    return 0.5 * z * (1.0 + lax.erf(z * 0.7071067811865476))


def _moe_body(sum_ref, cnt_ref, gate_ref, w1_ref, b1_ref, w2_ref, b2_ref,
              w3_ref, b3_ref, mol_ref, hw1_ref, hb1_ref, hw2_ref, hb2_ref,
              preds_ref, lb_ref):
    fs = sum_ref[...]
    cnt = cnt_ref[:, 0:1]
    femb = fs / jnp.maximum(cnt, 1.0)

    logits = jnp.dot(femb, gate_ref[...], precision=_PREC_REF,
                     preferred_element_type=jnp.float32)
    iota = lax.broadcasted_iota(jnp.int32, (_F, _NE), 1)
    v1 = jnp.max(logits, axis=1, keepdims=True)
    i1 = jnp.min(jnp.where(logits == v1, iota, _NE), axis=1, keepdims=True)
    l2 = jnp.where(iota == i1, -jnp.inf, logits)
    v2 = jnp.max(l2, axis=1, keepdims=True)
    i2 = jnp.min(jnp.where(l2 == v2, iota, _NE), axis=1, keepdims=True)
    e2 = jnp.exp(v2 - v1)
    denom = 1.0 + e2
    wts = (jnp.where(iota == i1, 1.0, 0.0)
           + jnp.where(iota == i2, e2, 0.0)) / denom
    load = jnp.sum(wts, axis=0, keepdims=True) / float(_F)
    lb_ref[...] = jnp.reshape(float(_NE) * jnp.sum(load * load), (1, 1))

    moe = jnp.zeros((_F, _D), jnp.float32)
    for e in range(_NE):
        z = jnp.dot(femb, w1_ref[e], precision=_PREC_REF,
                    preferred_element_type=jnp.float32) + b1_ref[e][None, :]
        z = _gelu(z)
        z = jnp.dot(z, w2_ref[e], precision=_PREC_REF,
                    preferred_element_type=jnp.float32) + b2_ref[e][None, :]
        z = _gelu(z)
        z = jnp.dot(z, w3_ref[e], precision=_PREC_REF,
                    preferred_element_type=jnp.float32) + b3_ref[e][None, :]
        moe = moe + wts[:, e:e + 1] * z

    iota_m = lax.broadcasted_iota(jnp.int32, (_F, _M), 1)
    sel = jnp.where(mol_ref[...] == iota_m, 1.0, 0.0)
    msum = lax.dot_general(sel, moe, (((0,), (0,)), ((), ())),
                           precision=_PREC, preferred_element_type=jnp.float32)
    ones_col = jnp.ones((_F, 1), jnp.float32)
    mcnt = lax.dot_general(sel, ones_col, (((0,), (0,)), ((), ())),
                           precision=_PREC, preferred_element_type=jnp.float32)
    memb = msum / jnp.maximum(mcnt, 1.0)

    for t in range(_NT):
        z = jnp.dot(memb, hw1_ref[t], precision=_PREC_REF,
                    preferred_element_type=jnp.float32) + hb1_ref[t][None, :]
        z = jnp.maximum(z, 0.0)
        p = jnp.dot(z, hw2_ref[t], precision=_PREC_REF,
                    preferred_element_type=jnp.float32) + hb2_ref[t][None, :]
        preds_ref[:, t:t + 1] = p


_moe_call = pl.pallas_call(
    _moe_body,
    out_shape=(jax.ShapeDtypeStruct((_M, _NT), jnp.float32),
               jax.ShapeDtypeStruct((1, 1), jnp.float32)),
)


def kernel(x, edge_index, batch, mol_idx, gin_W1, gin_b1, gin_W2, gin_b2,
           gin_eps, gate_W, exp_W1, exp_b1, exp_W2, exp_b2, exp_W3, exp_b3,
           head_W1, head_b1, head_W2, head_b2):
    idx4 = edge_index.astype(jnp.int32).reshape(2, _NS, _ECH, _EC)
    zh = jnp.zeros((_NPT, _DH), jnp.float32)
    h, hl, hr = x, x[:, :_DH], x[:, _DH:]
    for i in range(3):
        agg = _make_edge_agg()(hl, hr, idx4, zh)
        h, hl, hr = _gin_call(gin_eps[i].reshape(1, 1), h, agg[0], agg[1],
                              gin_W1[i], gin_b1[i].reshape(1, _D),
                              gin_W2[i], gin_b2[i].reshape(1, _D))
    hp = jnp.pad(h, ((0, _NP - _N), (0, 0)))
    bidx = jnp.pad(batch.astype(jnp.int32), (0, _NP - _N),
                   constant_values=_F).reshape(_NS, _PCH, _PC)
    onesv = jnp.ones((_PC, _CW), jnp.float32)
    zs = jnp.zeros((_FAT, _D), jnp.float32)
    fsum, fcnt = _make_pool()(hp, bidx, onesv, zs)
    preds, lb = _moe_call(fsum, fcnt, gate_W, exp_W1, exp_b1, exp_W2, exp_b2,
                          exp_W3, exp_b3, mol_idx.astype(jnp.int32).reshape(_F, 1),
                          head_W1, head_b1, head_W2, head_b2)
    return preds, lb[0, 0]
